# Initial kernel scaffold; baseline (speedup 1.0000x reference)
import jax
import jax.numpy as jnp
from jax.experimental import pallas as pl


def kernel(x, edge_index, W1, b1, W2, b2, Wc, bc, h_other):
    raise NotImplementedError("write your pallas kernel here")


# speedup vs baseline: 34.0899x; 34.0899x over previous
import functools

import jax
import jax.numpy as jnp
from jax import lax
from jax.experimental import pallas as pl
from jax.experimental.pallas import tpu as pltpu
from jax.experimental.pallas import tpu_sc as plsc

N = 10000
E = 320000
D = 128
NW = 32
EPT = E // NW
CH = 80
NCH = EPT // CH
NP_ = 10240
RPT = NP_ // 16
ZR = 128

_mesh = plsc.VectorSubcoreMesh(core_axis_name="c", subcore_axis_name="s")
_sc_params = pltpu.CompilerParams(needs_layout_passes=False)


def _zero_1d(ref, n):
    z = jnp.zeros((16,), jnp.float32)

    def body(i, _):
        ref[pl.ds(i * 16, 16)] = z
        return 0

    lax.fori_loop(0, n // 16, body, 0)


def _zero_2d(ref, rows):
    z = jnp.zeros((16,), jnp.float32)

    def body(i, _):
        for k in range(D // 16):
            ref[i, pl.ds(k * 16, 16)] = z
        return 0

    lax.fori_loop(0, rows, body, 0)


@functools.partial(
    pl.kernel,
    out_type=jax.ShapeDtypeStruct((NW, 1, N), jnp.float32),
    mesh=_mesh,
    compiler_params=_sc_params,
    scratch_types=[
        pltpu.VMEM((NCH, CH), jnp.int32),
        pltpu.VMEM((N,), jnp.float32),
    ],
)
def _sc_deg(dst_hbm, out_hbm, dst_v, deg_v):
    wid = lax.axis_index("s") * 2 + lax.axis_index("c")
    pltpu.sync_copy(dst_hbm.at[wid], dst_v)
    _zero_1d(deg_v, N)
    ones = jnp.ones((16,), jnp.float32)

    def body(j, _):
        for k in range(CH // 16):
            idx = dst_v[j, pl.ds(k * 16, 16)]
            plsc.addupdate_scatter(deg_v, [idx], ones)
        return 0

    lax.fori_loop(0, NCH, body, 0)
    pltpu.sync_copy(deg_v, out_hbm.at[wid, 0])


@functools.partial(
    pl.kernel,
    out_type=jax.ShapeDtypeStruct((NW, 1, N), jnp.float32),
    mesh=_mesh,
    compiler_params=_sc_params,
    scratch_types=[
        pltpu.VMEM((NCH, CH), jnp.int32),
        pltpu.VMEM((NCH, CH), jnp.int32),
        pltpu.VMEM((N,), jnp.float32),
        pltpu.VMEM((N,), jnp.float32),
    ],
)
def _sc_sacc(src_hbm, dst_hbm, dinv_hbm, s_out, src_v, dst_v, dinv_v, sacc_v):
    wid = lax.axis_index("s") * 2 + lax.axis_index("c")
    pltpu.sync_copy(src_hbm.at[wid], src_v)
    pltpu.sync_copy(dst_hbm.at[wid], dst_v)
    pltpu.sync_copy(dinv_hbm, dinv_v)
    _zero_1d(sacc_v, N)

    def sbody(j, _):
        for k in range(CH // 16):
            d16 = dst_v[j, pl.ds(k * 16, 16)]
            s16 = src_v[j, pl.ds(k * 16, 16)]
            vals = plsc.load_gather(dinv_v, [d16])
            plsc.addupdate_scatter(sacc_v, [s16], vals)
        return 0

    lax.fori_loop(0, NCH, sbody, 0)
    pltpu.sync_copy(sacc_v, s_out.at[wid, 0])


@functools.partial(
    pl.kernel,
    out_type=jax.ShapeDtypeStruct((2, NP_, D), jnp.float32),
    mesh=_mesh,
    compiler_params=_sc_params,
    scratch_types=[
        pltpu.VMEM((NCH, CH), jnp.int32),
        pltpu.VMEM((NCH, CH), jnp.int32),
        pltpu.VMEM((CH, D), jnp.float32),
        pltpu.VMEM_SHARED((NP_, D), jnp.float32),
        pltpu.SemaphoreType.DMA,
    ],
)
def _sc_agg(src_hbm, dst_hbm, g_hbm, agg_out, src_v, dst_v, rows_v, agg_sh, sem):
    cid = lax.axis_index("c")
    sid = lax.axis_index("s")
    wid = sid * 2 + cid
    pltpu.sync_copy(src_hbm.at[wid], src_v)
    pltpu.sync_copy(dst_hbm.at[wid], dst_v)
    _zero_2d(rows_v, CH)
    base = sid * RPT
    for t in range(RPT // CH):
        pltpu.sync_copy(rows_v, agg_sh.at[pl.ds(base + t * CH, CH)])
    plsc.subcore_barrier()

    def gbody(j, _):
        pltpu.async_copy(g_hbm.at[src_v.at[j]], rows_v, sem).wait()
        pltpu.sync_copy(rows_v, agg_sh.at[dst_v.at[j]], add=True)
        return 0

    lax.fori_loop(0, NCH, gbody, 0)
    plsc.subcore_barrier()
    for t in range(RPT // CH):
        r0 = base + t * CH
        pltpu.sync_copy(agg_sh.at[pl.ds(r0, CH)], rows_v)
        pltpu.sync_copy(rows_v, agg_out.at[cid, pl.ds(r0, CH)])


def _tc_mm_body(x_ref, w_ref, h_ref):
    h_ref[...] = jnp.dot(x_ref[...], w_ref[...],
                         preferred_element_type=jnp.float32)


def _tc_prep_body(parts_ref, h_ref, dinv_ref, g_ref):
    deg = jnp.sum(parts_ref[...], axis=0, keepdims=True) + 1.0
    dinv = lax.rsqrt(jnp.maximum(deg, 1.0))
    dinv_ref[...] = dinv
    g_ref[...] = h_ref[...] * jnp.transpose(dinv)


def _tc_fin_body(agg0_ref, agg1_ref, s_ref, g_ref, dinv_ref, b1_ref,
                 w2_ref, b2_ref, wc_ref, bc_ref, hoth_ref, out_ref):
    dinv = dinv_ref[...]
    sacc = jnp.sum(s_ref[...], axis=0, keepdims=True)
    c = dinv * (dinv + sacc)
    aggraw = agg0_ref[...] + agg1_ref[...] + g_ref[...]
    h1 = jnp.maximum(jnp.transpose(dinv) * aggraw + b1_ref[...], 0.0)
    v = jnp.dot(c, h1, preferred_element_type=jnp.float32)
    mean = jnp.dot(v * (1.0 / N), w2_ref[...],
                   preferred_element_type=jnp.float32) + b2_ref[...]
    z = jnp.concatenate([mean, hoth_ref[...]], axis=1)
    out_ref[...] = jnp.dot(z, wc_ref[...],
                           preferred_element_type=jnp.float32) + bc_ref[...]


def kernel(x, edge_index, W1, b1, W2, b2, Wc, bc, h_other):
    src = edge_index[0].astype(jnp.int32).reshape(NW, NCH, CH)
    dst = edge_index[1].astype(jnp.int32).reshape(NW, NCH, CH)

    deg_parts = _sc_deg(dst)

    h = pl.pallas_call(
        _tc_mm_body,
        out_shape=jax.ShapeDtypeStruct((N, D), jnp.float32),
    )(x, W1)

    dinv, g = pl.pallas_call(
        _tc_prep_body,
        out_shape=[
            jax.ShapeDtypeStruct((1, N), jnp.float32),
            jax.ShapeDtypeStruct((N, D), jnp.float32),
        ],
    )(deg_parts.reshape(NW, N), h)

    s_parts = _sc_sacc(src, dst, dinv.reshape(N)).reshape(NW, N)
    agg_parts = _sc_agg(src, dst, g)[:, :N, :]

    logits = pl.pallas_call(
        _tc_fin_body,
        out_shape=jax.ShapeDtypeStruct((1, 3), jnp.float32),
    )(agg_parts[0], agg_parts[1], s_parts, g, dinv,
      b1.reshape(1, D), W2, b2.reshape(1, D), Wc, bc.reshape(1, 3), h_other)
    return logits
